# baseline (speedup 1.0000x reference)
import functools
import math

import jax
import jax.numpy as jnp
import numpy as np
from jax import lax
from jax.experimental import pallas as pl
from jax.experimental.pallas import tpu as pltpu
from jax.sharding import Mesh, PartitionSpec as P

try:
    from jax import shard_map as _shard_map

    def _smap(f, mesh, in_specs, out_specs):
        return _shard_map(f, mesh=mesh, in_specs=in_specs, out_specs=out_specs,
                          check_vma=False)
except ImportError:
    from jax.experimental.shard_map import shard_map as _shard_map

    def _smap(f, mesh, in_specs, out_specs):
        return _shard_map(f, mesh=mesh, in_specs=in_specs, out_specs=out_specs,
                          check_rep=False)

NUM_HEADS = 12
LN_EPS = 1e-5


def _mha_ln_kernel(x_ref, w_ref, b_ref, g_ref, bt_ref, out_ref, *, num_heads):
    xf = x_ref[0]
    xb = xf.astype(jnp.bfloat16)

    S, E = xf.shape
    Dh = E // num_heads

    qkv = jnp.dot(xb, w_ref[...], preferred_element_type=jnp.float32) + b_ref[...]
    qkb = qkv[:, :2 * E].astype(jnp.bfloat16)
    v = qkv[:, 2 * E:]

    ctx_parts = []
    for h in range(num_heads):
        qh = qkb[:, h * Dh:(h + 1) * Dh]
        kh = qkb[:, E + h * Dh:E + (h + 1) * Dh]
        vh = v[:, h * Dh:(h + 1) * Dh]
        s = lax.dot_general(qh, kh, dimension_numbers=(((1,), (1,)), ((), ())),
                            preferred_element_type=jnp.float32)
        m = jnp.max(s, axis=-1, keepdims=True)
        p = jnp.exp(s - m)
        denom = jnp.sum(p, axis=-1, keepdims=True)
        ctx_h = jnp.dot(p, vh, preferred_element_type=jnp.float32)
        ctx_parts.append(ctx_h * pl.reciprocal(denom, approx=True))
    ctx = jnp.concatenate(ctx_parts, axis=-1)

    y = 4.0 * xf + ctx
    mean = jnp.mean(y, axis=-1, keepdims=True)
    c = y - mean
    var = jnp.mean(c * c, axis=-1, keepdims=True)
    inv = lax.rsqrt(var + LN_EPS)
    out_ref[0] = (c * inv) * g_ref[...] + bt_ref[...]


def _mha_block(x, w, b, g, bt):
    B, S, E = x.shape
    row_spec = pl.BlockSpec((1, S, E), lambda i: (i, 0, 0))
    w_spec = pl.BlockSpec((E, 3 * E), lambda i: (0, 0))
    b_spec = pl.BlockSpec((1, 3 * E), lambda i: (0, 0))
    vec_spec = pl.BlockSpec((1, E), lambda i: (0, 0))

    return pl.pallas_call(
        functools.partial(_mha_ln_kernel, num_heads=NUM_HEADS),
        out_shape=jax.ShapeDtypeStruct((B, S, E), jnp.float32),
        grid=(B,),
        in_specs=[row_spec, w_spec, b_spec, vec_spec, vec_spec],
        out_specs=row_spec,
        compiler_params=pltpu.CompilerParams(
            dimension_semantics=("parallel",),
            vmem_limit_bytes=64 * 1024 * 1024,
        ),
    )(x, w, b, g, bt)


def kernel(x, wq, bq, wk, bk, wv, bv, gamma, beta):
    B, S, E = x.shape
    scale = 1.0 / math.sqrt(E // NUM_HEADS)

    w = jnp.concatenate([wq * scale, wk, wv], axis=1).astype(jnp.bfloat16)
    b = jnp.concatenate([bq * scale, bk, bv]).reshape(1, 3 * E)
    g = gamma.reshape(1, E)
    bt = beta.reshape(1, E)

    devs = jax.devices()
    n = len(devs)
    while n > 1 and B % n:
        n -= 1
    if n == 1:
        return _mha_block(x, w, b, g, bt)

    mesh = Mesh(np.array(devs[:n]), ("d",))
    f = _smap(_mha_block, mesh,
              (P("d"), P(), P(), P(), P()), P("d"))
    return f(x, w, b, g, bt)


# speedup vs baseline: 2.4989x; 2.4989x over previous
import functools
import math

import jax
import jax.numpy as jnp
from jax import lax
from jax.experimental import pallas as pl
from jax.experimental.pallas import tpu as pltpu

NUM_HEADS = 12
LN_EPS = 1e-5


def _mha_ln_kernel(x_ref, w_ref, b_ref, g_ref, bt_ref, out_ref, *, num_heads):
    xf = x_ref[0]
    xb = xf.astype(jnp.bfloat16)

    S, E = xf.shape
    Dh = E // num_heads

    qkv = jnp.dot(xb, w_ref[...], preferred_element_type=jnp.float32) + b_ref[...]
    qkb = qkv[:, :2 * E].astype(jnp.bfloat16)
    v = qkv[:, 2 * E:]

    ctx_parts = []
    for h in range(num_heads):
        qh = qkb[:, h * Dh:(h + 1) * Dh]
        kh = qkb[:, E + h * Dh:E + (h + 1) * Dh]
        vh = v[:, h * Dh:(h + 1) * Dh]
        s = lax.dot_general(qh, kh, dimension_numbers=(((1,), (1,)), ((), ())),
                            preferred_element_type=jnp.float32)
        m = jnp.max(s, axis=-1, keepdims=True)
        p = jnp.exp(s - m)
        denom = jnp.sum(p, axis=-1, keepdims=True)
        ctx_h = jnp.dot(p, vh, preferred_element_type=jnp.float32)
        ctx_parts.append(ctx_h * pl.reciprocal(denom, approx=True))
    ctx = jnp.concatenate(ctx_parts, axis=-1)

    y = 4.0 * xf + ctx
    mean = jnp.mean(y, axis=-1, keepdims=True)
    c = y - mean
    var = jnp.mean(c * c, axis=-1, keepdims=True)
    inv = lax.rsqrt(var + LN_EPS)
    out_ref[0] = (c * inv) * g_ref[...] + bt_ref[...]


def _mha_block(x, w, b, g, bt):
    B, S, E = x.shape
    row_spec = pl.BlockSpec((1, S, E), lambda i: (i, 0, 0))
    w_spec = pl.BlockSpec((E, 3 * E), lambda i: (0, 0))
    b_spec = pl.BlockSpec((1, 3 * E), lambda i: (0, 0))
    vec_spec = pl.BlockSpec((1, E), lambda i: (0, 0))

    return pl.pallas_call(
        functools.partial(_mha_ln_kernel, num_heads=NUM_HEADS),
        out_shape=jax.ShapeDtypeStruct((B, S, E), jnp.float32),
        grid=(B,),
        in_specs=[row_spec, w_spec, b_spec, vec_spec, vec_spec],
        out_specs=row_spec,
        compiler_params=pltpu.CompilerParams(
            dimension_semantics=("parallel",),
            vmem_limit_bytes=64 * 1024 * 1024,
        ),
    )(x, w, b, g, bt)


def kernel(x, wq, bq, wk, bk, wv, bv, gamma, beta):
    B, S, E = x.shape
    scale = 1.0 / math.sqrt(E // NUM_HEADS)

    w = jnp.concatenate([wq * scale, wk, wv], axis=1).astype(jnp.bfloat16)
    b = jnp.concatenate([bq * scale, bk, bv]).reshape(1, 3 * E)
    g = gamma.reshape(1, E)
    bt = beta.reshape(1, E)

    return _mha_block(x, w, b, g, bt)


# speedup vs baseline: 2.6679x; 1.0676x over previous
import functools
import math

import jax
import jax.numpy as jnp
from jax import lax
from jax.experimental import pallas as pl
from jax.experimental.pallas import tpu as pltpu

NUM_HEADS = 12
LN_EPS = 1e-5


def _mha_ln_kernel(x_ref, w_ref, b_ref, g_ref, bt_ref, out_ref, *, num_heads):
    xf = x_ref[0]
    xb = xf.astype(jnp.bfloat16)

    S, E = xf.shape
    Dh = E // num_heads

    qkv = jnp.dot(xb, w_ref[...], preferred_element_type=jnp.float32) + b_ref[...]
    qkb = qkv[:, :2 * E].astype(jnp.bfloat16)
    v = qkv[:, 2 * E:]

    s_list = []
    for h in range(num_heads):
        qh = qkb[:, h * Dh:(h + 1) * Dh]
        kh = qkb[:, E + h * Dh:E + (h + 1) * Dh]
        s_list.append(
            lax.dot_general(qh, kh, dimension_numbers=(((1,), (1,)), ((), ())),
                            preferred_element_type=jnp.float32))

    ctx_parts = []
    for h in range(num_heads):
        s = s_list[h]
        vh = v[:, h * Dh:(h + 1) * Dh]
        m = jnp.max(s, axis=-1, keepdims=True)
        p = jnp.exp(s - m)
        denom = jnp.sum(p, axis=-1, keepdims=True)
        ctx_h = jnp.dot(p, vh, preferred_element_type=jnp.float32)
        ctx_parts.append(ctx_h * pl.reciprocal(denom, approx=True))
    ctx = jnp.concatenate(ctx_parts, axis=-1)

    y = 4.0 * xf + ctx
    mean = jnp.mean(y, axis=-1, keepdims=True)
    c = y - mean
    var = jnp.mean(c * c, axis=-1, keepdims=True)
    inv = lax.rsqrt(var + LN_EPS)
    out_ref[0] = (c * inv) * g_ref[...] + bt_ref[...]


def _mha_block(x, w, b, g, bt):
    B, S, E = x.shape
    row_spec = pl.BlockSpec((1, S, E), lambda i: (i, 0, 0))
    w_spec = pl.BlockSpec((E, 3 * E), lambda i: (0, 0))
    b_spec = pl.BlockSpec((1, 3 * E), lambda i: (0, 0))
    vec_spec = pl.BlockSpec((1, E), lambda i: (0, 0))

    return pl.pallas_call(
        functools.partial(_mha_ln_kernel, num_heads=NUM_HEADS),
        out_shape=jax.ShapeDtypeStruct((B, S, E), jnp.float32),
        grid=(B,),
        in_specs=[row_spec, w_spec, b_spec, vec_spec, vec_spec],
        out_specs=row_spec,
        compiler_params=pltpu.CompilerParams(
            dimension_semantics=("parallel",),
            vmem_limit_bytes=64 * 1024 * 1024,
        ),
    )(x, w, b, g, bt)


def kernel(x, wq, bq, wk, bk, wv, bv, gamma, beta):
    B, S, E = x.shape
    scale = 1.0 / math.sqrt(E // NUM_HEADS)

    w = jnp.concatenate([wq * scale, wk, wv], axis=1).astype(jnp.bfloat16)
    b = jnp.concatenate([bq * scale, bk, bv]).reshape(1, 3 * E)
    g = gamma.reshape(1, E)
    bt = beta.reshape(1, E)

    return _mha_block(x, w, b, g, bt)
